# baseline (speedup 1.0000x reference)
import functools

import jax
import jax.numpy as jnp
from jax import lax
from jax.experimental import pallas as pl
from jax.experimental.pallas import tpu as pltpu
from jax.experimental.pallas import tpu_sc as plsc

EPS_ = 1e-06
B_ = 8
H_ = 512
W_ = 512
HW_ = H_ * W_
M_ = 1000
MP_ = 1024
NC_ = 2
NS_ = 16
LANES_ = 16
PER_W_ = B_ * HW_ // (NC_ * NS_)
WROWS_ = PER_W_ // W_
IN_ROWS_ = 8
N_IN_ = WROWS_ // IN_ROWS_
OUT_ROWS_ = 16
N_OUT_ = WROWS_ // OUT_ROWS_


def _sc_body(p_hbm, ids_hbm, census_hbm, out_hbm,
             hist, ratio, crow, quad, ist, pst,
             ib0, ib1, pb0, pb1, ob0, ob1,
             si0, si1, sp0, sp1, sc, so0, so1, shared):
    c = lax.axis_index("c")
    s = lax.axis_index("s")
    batch = c * 4 + s // 4
    row0 = (s % 4) * WROWS_

    ib = (ib0, ib1)
    pb = (pb0, pb1)
    ob = (ob0, ob1)
    si = (si0, si1)
    sp = (sp0, sp1)
    so = (so0, so1)

    ccp = pltpu.async_copy(census_hbm.at[batch], crow, sc)

    def start_in(k):
        bi = k % 2
        r = row0 + k * IN_ROWS_
        c1 = pltpu.async_copy(ids_hbm.at[batch, pl.ds(r, IN_ROWS_), :],
                              ib[bi], si[bi])
        c2 = pltpu.async_copy(p_hbm.at[batch, 0, pl.ds(r, IN_ROWS_), :],
                              pb[bi], sp[bi])
        return (c1, c2)

    cps = [start_in(0), None]
    trc = jax.named_scope

    zeros16 = jnp.zeros((LANES_,), jnp.float32)

    @plsc.parallel_loop(0, LANES_ * MP_ // 16, unroll=8)
    def _(i):
        hist[pl.ds(i * 16, 16)] = zeros16

    lane_i = lax.iota(jnp.int32, 16)

    for k in range(N_IN_):
      with trc(f"p1_chunk{k}"):
        bi = k % 2
        if k + 1 < N_IN_:
            cps[(k + 1) % 2] = start_in(k + 1)
        for d in cps[bi]:
            d.wait()
        ibk, pbk = ib[bi], pb[bi]

        @plsc.parallel_loop(0, IN_ROWS_ * W_ // 32, unroll=8)
        def _(i):
            r = i >> 4
            c0 = (i & 15) * 32
            w = k * (IN_ROWS_ * W_ // 2) + i * 16
            ia = ibk[r, pl.ds(c0, 16)]
            ibb = ibk[r, pl.ds(c0 + 16, 16)]
            pa = pbk[r, pl.ds(c0, 16)]
            pbv = pbk[r, pl.ds(c0 + 16, 16)]
            plsc.addupdate_scatter(hist, [ia * 16 + lane_i], pa)
            plsc.addupdate_scatter(hist, [ibb * 16 + lane_i], pbv)
            ist[pl.ds(w, 16)] = plsc.bitcast(
                plsc.pack(ia, ibb, format=plsc.PackFormat.INTERLEAVED),
                jnp.int32)
            pst[pl.ds(w, 16)] = plsc.bitcast(
                plsc.pack(pa, pbv, format=plsc.PackFormat.INTERLEAVED),
                jnp.float32)

    diag = [lane_i * 16 + ((lane_i + m) & 15) for m in range(LANES_)]

    with trc("p2_lanered"):
        @plsc.parallel_loop(0, MP_ // 16, unroll=2)
        def _(j):
            base = j * 256
            acc = plsc.load_gather(hist, [base + diag[0]])
            for m in range(1, LANES_):
                acc = acc + plsc.load_gather(hist, [base + diag[m]])
            ratio[pl.ds(j * 16, 16)] = acc

        pltpu.sync_copy(ratio, shared.at[s])
    with trc("p2_barrier"):
        plsc.subcore_barrier()

    with trc("p2_combine"):
        gb = (s // 4) * 4
        pltpu.sync_copy(shared.at[pl.ds(gb, 4)], quad)
        ccp.wait()

        @plsc.parallel_loop(0, MP_ // 16, unroll=2)
        def _(j):
            sl = pl.ds(j * 16, 16)
            stot = quad[0, sl] + quad[1, sl] + quad[2, sl] + quad[3, sl] + EPS_
            ratio[sl] = crow[sl] / stot

    ocp = [None, None]
    for k in range(N_OUT_):
      with trc(f"p3_chunk{k}"):
        bi = k % 2
        if ocp[bi] is not None:
            ocp[bi].wait()
        obk = ob[bi]

        @plsc.parallel_loop(0, OUT_ROWS_ * W_ // 32, unroll=8)
        def _(i):
            r = i >> 4
            c0 = (i & 15) * 32
            w = k * (OUT_ROWS_ * W_ // 2) + i * 16
            ia, ibb = plsc.unpack(
                plsc.bitcast(ist[pl.ds(w, 16)], jnp.int16),
                format=plsc.PackFormat.INTERLEAVED)
            pa, pbv = plsc.unpack(
                plsc.bitcast(pst[pl.ds(w, 16)], jnp.bfloat16),
                format=plsc.PackFormat.INTERLEAVED)
            ra = plsc.load_gather(ratio, [ia])
            rb = plsc.load_gather(ratio, [ibb])
            obk[r, pl.ds(c0, 16)] = pa * ra
            obk[r, pl.ds(c0 + 16, 16)] = pbv * rb

        ocp[bi] = pltpu.async_copy(
            obk, out_hbm.at[batch, 0, pl.ds(row0 + k * OUT_ROWS_, OUT_ROWS_), :],
            so[bi])

    with trc("drain"):
        ocp[0].wait()
        ocp[1].wait()


@jax.jit
def _census_sc(P_raw, ids, census_pad):
    mesh = plsc.VectorSubcoreMesh(core_axis_name="c", subcore_axis_name="s")
    kfn = functools.partial(
        pl.kernel, mesh=mesh,
        out_type=jax.ShapeDtypeStruct((B_, 1, H_, W_), jnp.float32),
        scratch_types=[
            pltpu.VMEM((LANES_ * MP_,), jnp.float32),
            pltpu.VMEM((MP_,), jnp.float32),
            pltpu.VMEM((MP_,), jnp.float32),
            pltpu.VMEM((4, MP_), jnp.float32),
            pltpu.VMEM((PER_W_ // 2,), jnp.int32),
            pltpu.VMEM((PER_W_ // 2,), jnp.float32),
            pltpu.VMEM((IN_ROWS_, W_), jnp.int32),
            pltpu.VMEM((IN_ROWS_, W_), jnp.int32),
            pltpu.VMEM((IN_ROWS_, W_), jnp.float32),
            pltpu.VMEM((IN_ROWS_, W_), jnp.float32),
            pltpu.VMEM((OUT_ROWS_, W_), jnp.float32),
            pltpu.VMEM((OUT_ROWS_, W_), jnp.float32),
            pltpu.SemaphoreType.DMA,
            pltpu.SemaphoreType.DMA,
            pltpu.SemaphoreType.DMA,
            pltpu.SemaphoreType.DMA,
            pltpu.SemaphoreType.DMA,
            pltpu.SemaphoreType.DMA,
            pltpu.SemaphoreType.DMA,
            pltpu.VMEM_SHARED((NS_, MP_), jnp.float32),
        ],
        compiler_params=pltpu.CompilerParams(needs_layout_passes=False),
    )(_sc_body)
    return kfn(P_raw, ids, census_pad)


def kernel(P_raw, admin_ids, census_totals):
    census_pad = jnp.zeros((B_, MP_), jnp.float32).at[:, :M_].set(
        census_totals)
    return _census_sc(P_raw, admin_ids, census_pad)


# speedup vs baseline: 1.0571x; 1.0571x over previous
import functools

import jax
import jax.numpy as jnp
from jax import lax
from jax.experimental import pallas as pl
from jax.experimental.pallas import tpu as pltpu
from jax.experimental.pallas import tpu_sc as plsc

EPS_ = 1e-06
B_ = 8
H_ = 512
W_ = 512
HW_ = H_ * W_
M_ = 1000
MP_ = 1024
NC_ = 2
NS_ = 16
LANES_ = 16
PER_W_ = B_ * HW_ // (NC_ * NS_)
WROWS_ = PER_W_ // W_
IN_ROWS_ = 8
N_IN_ = WROWS_ // IN_ROWS_
OUT_ROWS_ = 16
N_OUT_ = WROWS_ // OUT_ROWS_


def _sc_body(p_hbm, ids_hbm, census_hbm, out_hbm,
             hist, ratio, crow, quad, ist, pst,
             ib0, ib1, ib2, pb0, pb1, pb2, ob0, ob1,
             si0, si1, si2, sp0, sp1, sp2, sc, so0, so1, shared):
    c = lax.axis_index("c")
    s = lax.axis_index("s")
    batch = c * 4 + s // 4
    row0 = (s % 4) * WROWS_

    ib = (ib0, ib1, ib2)
    pb = (pb0, pb1, pb2)
    ob = (ob0, ob1)
    si = (si0, si1, si2)
    sp = (sp0, sp1, sp2)
    so = (so0, so1)

    ccp = pltpu.async_copy(census_hbm.at[batch], crow, sc)

    def start_in(k):
        bi = k % 3
        r = row0 + k * IN_ROWS_
        c1 = pltpu.async_copy(ids_hbm.at[batch, pl.ds(r, IN_ROWS_), :],
                              ib[bi], si[bi])
        c2 = pltpu.async_copy(p_hbm.at[batch, 0, pl.ds(r, IN_ROWS_), :],
                              pb[bi], sp[bi])
        return (c1, c2)

    cps = [start_in(0), start_in(1), None]

    zeros16 = jnp.zeros((LANES_,), jnp.float32)

    @plsc.parallel_loop(0, LANES_ * MP_ // 16, unroll=8)
    def _(i):
        hist[pl.ds(i * 16, 16)] = zeros16

    lane_i = lax.iota(jnp.int32, 16)

    for k in range(N_IN_):
        bi = k % 3
        if k + 2 < N_IN_:
            cps[(k + 2) % 3] = start_in(k + 2)
        for d in cps[bi]:
            d.wait()
        ibk, pbk = ib[bi], pb[bi]

        @plsc.parallel_loop(0, IN_ROWS_ * W_ // 32, unroll=8)
        def _(i):
            r = i >> 4
            c0 = (i & 15) * 32
            w = k * (IN_ROWS_ * W_ // 2) + i * 16
            ia = ibk[r, pl.ds(c0, 16)]
            ibb = ibk[r, pl.ds(c0 + 16, 16)]
            pa = pbk[r, pl.ds(c0, 16)]
            pbv = pbk[r, pl.ds(c0 + 16, 16)]
            plsc.addupdate_scatter(hist, [ia * 16 + lane_i], pa)
            plsc.addupdate_scatter(hist, [ibb * 16 + lane_i], pbv)
            ist[pl.ds(w, 16)] = plsc.bitcast(
                plsc.pack(ia, ibb, format=plsc.PackFormat.INTERLEAVED),
                jnp.int32)
            pst[pl.ds(w, 16)] = plsc.bitcast(
                plsc.pack(pa, pbv, format=plsc.PackFormat.INTERLEAVED),
                jnp.float32)

    diag = [lane_i * 16 + ((lane_i + m) & 15) for m in range(LANES_)]

    @plsc.parallel_loop(0, MP_ // 16, unroll=2)
    def _(j):
        base = j * 256
        acc = plsc.load_gather(hist, [base + diag[0]])
        for m in range(1, LANES_):
            acc = acc + plsc.load_gather(hist, [base + diag[m]])
        ratio[pl.ds(j * 16, 16)] = acc

    pltpu.sync_copy(ratio, shared.at[s])
    plsc.subcore_barrier()

    gb = (s // 4) * 4
    pltpu.sync_copy(shared.at[pl.ds(gb, 4)], quad)
    ccp.wait()

    @plsc.parallel_loop(0, MP_ // 16, unroll=2)
    def _(j):
        sl = pl.ds(j * 16, 16)
        stot = quad[0, sl] + quad[1, sl] + quad[2, sl] + quad[3, sl] + EPS_
        ratio[sl] = crow[sl] / stot

    ocp = [None, None]
    for k in range(N_OUT_):
        bi = k % 2
        if ocp[bi] is not None:
            ocp[bi].wait()
        obk = ob[bi]

        @plsc.parallel_loop(0, OUT_ROWS_ * W_ // 32, unroll=8)
        def _(i):
            r = i >> 4
            c0 = (i & 15) * 32
            w = k * (OUT_ROWS_ * W_ // 2) + i * 16
            ia, ibb = plsc.unpack(
                plsc.bitcast(ist[pl.ds(w, 16)], jnp.int16),
                format=plsc.PackFormat.INTERLEAVED)
            pa, pbv = plsc.unpack(
                plsc.bitcast(pst[pl.ds(w, 16)], jnp.bfloat16),
                format=plsc.PackFormat.INTERLEAVED)
            ra = plsc.load_gather(ratio, [ia])
            rb = plsc.load_gather(ratio, [ibb])
            obk[r, pl.ds(c0, 16)] = pa * ra
            obk[r, pl.ds(c0 + 16, 16)] = pbv * rb

        ocp[bi] = pltpu.async_copy(
            obk, out_hbm.at[batch, 0, pl.ds(row0 + k * OUT_ROWS_, OUT_ROWS_), :],
            so[bi])

    ocp[0].wait()
    ocp[1].wait()


@jax.jit
def _census_sc(P_raw, ids, census_pad):
    mesh = plsc.VectorSubcoreMesh(core_axis_name="c", subcore_axis_name="s")
    kfn = functools.partial(
        pl.kernel, mesh=mesh,
        out_type=jax.ShapeDtypeStruct((B_, 1, H_, W_), jnp.float32),
        scratch_types=[
            pltpu.VMEM((LANES_ * MP_,), jnp.float32),
            pltpu.VMEM((MP_,), jnp.float32),
            pltpu.VMEM((MP_,), jnp.float32),
            pltpu.VMEM((4, MP_), jnp.float32),
            pltpu.VMEM((PER_W_ // 2,), jnp.int32),
            pltpu.VMEM((PER_W_ // 2,), jnp.float32),
            pltpu.VMEM((IN_ROWS_, W_), jnp.int32),
            pltpu.VMEM((IN_ROWS_, W_), jnp.int32),
            pltpu.VMEM((IN_ROWS_, W_), jnp.int32),
            pltpu.VMEM((IN_ROWS_, W_), jnp.float32),
            pltpu.VMEM((IN_ROWS_, W_), jnp.float32),
            pltpu.VMEM((IN_ROWS_, W_), jnp.float32),
            pltpu.VMEM((OUT_ROWS_, W_), jnp.float32),
            pltpu.VMEM((OUT_ROWS_, W_), jnp.float32),
            pltpu.SemaphoreType.DMA,
            pltpu.SemaphoreType.DMA,
            pltpu.SemaphoreType.DMA,
            pltpu.SemaphoreType.DMA,
            pltpu.SemaphoreType.DMA,
            pltpu.SemaphoreType.DMA,
            pltpu.SemaphoreType.DMA,
            pltpu.SemaphoreType.DMA,
            pltpu.SemaphoreType.DMA,
            pltpu.VMEM_SHARED((NS_, MP_), jnp.float32),
        ],
        compiler_params=pltpu.CompilerParams(needs_layout_passes=False),
    )(_sc_body)
    return kfn(P_raw, ids, census_pad)


def kernel(P_raw, admin_ids, census_totals):
    census_pad = jnp.zeros((B_, MP_), jnp.float32).at[:, :M_].set(
        census_totals)
    return _census_sc(P_raw, admin_ids, census_pad)
